# Initial kernel scaffold; baseline (speedup 1.0000x reference)
import functools

import jax
import jax.numpy as jnp
from jax import lax
from jax.experimental import pallas as pl
from jax.experimental.pallas import tpu as pltpu
from jax.experimental.pallas import tpu_sc as plsc

B = 4096
F = 26
D = 64
NW = 32
BPW = B // NW
CH = 4
NCHUNK = BPW // CH
ROWS = CH * F


def _fm_body(xi_hbm, fm1_hbm, fm2_hbm, bias_hbm, out_hbm,
             idx_v, rb0, rb1, eb0, eb1, out_v, bias_v, sem0, sem1):
    wid = lax.axis_index("s") * 2 + lax.axis_index("c")
    base = wid * BPW

    pltpu.sync_copy(xi_hbm.at[pl.ds(base * F, BPW * F)], idx_v)
    pltpu.sync_copy(bias_hbm, bias_v)
    bias_s = bias_v[0]

    rbufs = (rb0, rb1)
    ebufs = (eb0, eb1)
    sems = (sem0, sem1)

    def copies(c, p):
        idx_sl = idx_v.at[pl.ds(c * ROWS, ROWS)]
        return (
            pltpu.make_async_copy(fm2_hbm.at[idx_sl], rbufs[p], sems[p]),
            pltpu.make_async_copy(fm1_hbm.at[idx_sl], ebufs[p], sems[p]),
        )

    def start_chunk(c, p):
        for cp in copies(c, p):
            cp.start()

    def wait_chunk(c, p):
        for cp in copies(c, p):
            cp.wait()

    def compute_chunk(c, p):
        rb = rbufs[p]
        eb = ebufs[p]
        for r in range(CH):
            def f_body(f, carry):
                a0, a1, a2, a3, asq, e1 = carry
                row = r * F + f
                x0 = rb[row, pl.ds(0, 16)]
                x1 = rb[row, pl.ds(16, 16)]
                x2 = rb[row, pl.ds(32, 16)]
                x3 = rb[row, pl.ds(48, 16)]
                asq = asq + x0 * x0 + x1 * x1 + x2 * x2 + x3 * x3
                e1 = e1 + eb[row]
                return (a0 + x0, a1 + x1, a2 + x2, a3 + x3, asq, e1)

            z = jnp.zeros((16,), jnp.float32)
            a0, a1, a2, a3, asq, e1 = lax.fori_loop(
                0, F, f_body, (z, z, z, z, z, jnp.float32(0.0)))
            t = a0 * a0 + a1 * a1 + a2 * a2 + a3 * a3 - asq
            out_v[c * CH + r] = 0.5 * jnp.sum(t) + e1 + bias_s

    start_chunk(0, 0)

    def body(i, _):
        c0 = 2 * i
        c1 = 2 * i + 1
        start_chunk(c1, 1)
        wait_chunk(c0, 0)
        compute_chunk(c0, 0)

        @pl.when(i < NCHUNK // 2 - 1)
        def _():
            start_chunk(c0 + 2, 0)

        wait_chunk(c1, 1)
        compute_chunk(c1, 1)
        return 0

    lax.fori_loop(0, NCHUNK // 2, body, 0)

    pltpu.sync_copy(out_v, out_hbm.at[pl.ds(base, BPW)])


@jax.jit
def _fm_sc(xi_flat, fm1_flat, fm_2nd, bias):
    mesh = plsc.VectorSubcoreMesh(core_axis_name="c", subcore_axis_name="s")
    fn = functools.partial(
        pl.kernel,
        mesh=mesh,
        out_type=jax.ShapeDtypeStruct((B,), jnp.float32),
        scratch_types=[
            pltpu.VMEM((BPW * F,), jnp.int32),
            pltpu.VMEM((ROWS, D), jnp.float32),
            pltpu.VMEM((ROWS, D), jnp.float32),
            pltpu.VMEM((ROWS,), jnp.float32),
            pltpu.VMEM((ROWS,), jnp.float32),
            pltpu.VMEM((BPW,), jnp.float32),
            pltpu.VMEM((1,), jnp.float32),
            pltpu.SemaphoreType.DMA,
            pltpu.SemaphoreType.DMA,
        ],
    )(_fm_body)
    return fn(xi_flat, fm1_flat, fm_2nd, bias)


def kernel(xi, fm_1st, fm_2nd, bias):
    return _fm_sc(xi.reshape(-1), fm_1st.reshape(-1), fm_2nd, bias)


# speedup vs baseline: 1.8109x; 1.8109x over previous
import functools

import jax
import jax.numpy as jnp
from jax import lax
from jax.experimental import pallas as pl
from jax.experimental.pallas import tpu as pltpu
from jax.experimental.pallas import tpu_sc as plsc

B = 4096
F = 26
D = 64
NW = 32
BPW = B // NW
GR = 16
NG = BPW // GR
NDMA = 4
RPD = GR * F // NDMA
ROWS = GR * F


def _fm_body(xi_hbm, fm1_hbm, fm2_hbm, bias_hbm, out_hbm,
             idx_v, rb0, rb1, eb0, eb1, out_v, bias_v, sem0, sem1):
    wid = lax.axis_index("s") * 2 + lax.axis_index("c")
    base = wid * BPW

    pltpu.sync_copy(xi_hbm.at[pl.ds(base * F, BPW * F)], idx_v)
    pltpu.sync_copy(bias_hbm, bias_v.at[pl.ds(0, 1)])
    bias_s = bias_v[pl.ds(0, 16)][0]

    lane = lax.broadcasted_iota(jnp.int32, (16,), 0)
    tail_mask = lane < (F - 16)

    rbufs = (rb0, rb1)
    ebufs = (eb0, eb1)
    sems = (sem0, sem1)

    def copies(g, p):
        out = []
        for q in range(NDMA):
            idx_sl = idx_v.at[pl.ds(g * ROWS + q * RPD, RPD)]
            out.append(pltpu.make_async_copy(
                fm2_hbm.at[idx_sl], rbufs[p].at[pl.ds(q * RPD, RPD), :],
                sems[p]))
            out.append(pltpu.make_async_copy(
                fm1_hbm.at[idx_sl], ebufs[p].at[pl.ds(q * RPD, RPD)],
                sems[p]))
        return out

    def start_group(g, p):
        for cp in copies(g, p):
            cp.start()

    def wait_group(g, p):
        for cp in copies(g, p):
            cp.wait()

    def compute_group(g, p):
        rb = rbufs[p]
        eb = ebufs[p]

        def row_body(r, acc):
            def f_body(f, carry):
                a0, a1, a2, a3, asq = carry
                row = r * F + f
                x0 = rb[row, pl.ds(0, 16)]
                x1 = rb[row, pl.ds(16, 16)]
                x2 = rb[row, pl.ds(32, 16)]
                x3 = rb[row, pl.ds(48, 16)]
                asq = asq + x0 * x0 + x1 * x1 + x2 * x2 + x3 * x3
                return (a0 + x0, a1 + x1, a2 + x2, a3 + x3, asq)

            z = jnp.zeros((16,), jnp.float32)
            a0, a1, a2, a3, asq = lax.fori_loop(
                0, F, f_body, (z, z, z, z, z))
            t = a0 * a0 + a1 * a1 + a2 * a2 + a3 * a3 - asq
            e1a = eb[pl.ds(r * F, 16)]
            e1b = eb[pl.ds(r * F + 16, 16)]
            v = 0.5 * t + e1a + jnp.where(tail_mask, e1b, 0.0)
            s = jnp.sum(v)
            return jnp.where(lane == r, s, acc)

        acc = lax.fori_loop(0, GR, row_body, jnp.zeros((16,), jnp.float32))
        out_v[pl.ds(g * GR, GR)] = acc + bias_s

    start_group(0, 0)

    def body(i, _):
        g0 = 2 * i
        g1 = 2 * i + 1
        start_group(g1, 1)
        wait_group(g0, 0)
        compute_group(g0, 0)

        @pl.when(i < NG // 2 - 1)
        def _():
            start_group(g0 + 2, 0)

        wait_group(g1, 1)
        compute_group(g1, 1)
        return 0

    lax.fori_loop(0, NG // 2, body, 0)

    pltpu.sync_copy(out_v, out_hbm.at[pl.ds(base, BPW)])


@jax.jit
def _fm_sc(xi_flat, fm1_flat, fm_2nd, bias):
    mesh = plsc.VectorSubcoreMesh(core_axis_name="c", subcore_axis_name="s")
    fn = functools.partial(
        pl.kernel,
        mesh=mesh,
        compiler_params=pltpu.CompilerParams(
            needs_layout_passes=False, use_tc_tiling_on_sc=False),
        out_type=jax.ShapeDtypeStruct((B,), jnp.float32),
        scratch_types=[
            pltpu.VMEM((BPW * F,), jnp.int32),
            pltpu.VMEM((ROWS, D), jnp.float32),
            pltpu.VMEM((ROWS, D), jnp.float32),
            pltpu.VMEM((ROWS + 16,), jnp.float32),
            pltpu.VMEM((ROWS + 16,), jnp.float32),
            pltpu.VMEM((BPW,), jnp.float32),
            pltpu.VMEM((16,), jnp.float32),
            pltpu.SemaphoreType.DMA,
            pltpu.SemaphoreType.DMA,
        ],
    )(_fm_body)
    return fn(xi_flat, fm1_flat, fm_2nd, bias)


def kernel(xi, fm_1st, fm_2nd, bias):
    return _fm_sc(xi.reshape(-1), fm_1st.reshape(-1), fm_2nd, bias)


# speedup vs baseline: 1.8887x; 1.0429x over previous
import functools

import jax
import jax.numpy as jnp
from jax import lax
from jax.experimental import pallas as pl
from jax.experimental.pallas import tpu as pltpu
from jax.experimental.pallas import tpu_sc as plsc

B = 4096
F = 26
D = 64
NW = 32
BPW = B // NW
GR = 16
NG = BPW // GR
NDMA = 4
RPD = GR * F // NDMA
ROWS = GR * F


def _fm_body(xi_hbm, fm1_hbm, fm2_hbm, bias_hbm, out_hbm,
             idx_v, rb0, rb1, eb0, eb1, out_v, bias_v, sem0, sem1):
    wid = lax.axis_index("s") * 2 + lax.axis_index("c")
    base = wid * BPW

    pltpu.sync_copy(xi_hbm.at[pl.ds(base * F, BPW * F)], idx_v)
    pltpu.sync_copy(bias_hbm, bias_v.at[pl.ds(0, 1)])
    bias_s = bias_v[pl.ds(0, 16)][0]

    lane = lax.broadcasted_iota(jnp.int32, (16,), 0)
    tail_mask = lane < (F - 16)

    rbufs = (rb0, rb1)
    ebufs = (eb0, eb1)
    sems = (sem0, sem1)

    def copies(g, p):
        out = []
        for q in range(NDMA):
            idx_sl = idx_v.at[pl.ds(g * ROWS + q * RPD, RPD)]
            out.append(pltpu.make_async_copy(
                fm2_hbm.at[idx_sl], rbufs[p].at[pl.ds(q * RPD, RPD), :],
                sems[p]))
            out.append(pltpu.make_async_copy(
                fm1_hbm.at[idx_sl], ebufs[p].at[pl.ds(q * RPD, RPD)],
                sems[p]))
        return out

    def start_group(g, p):
        for cp in copies(g, p):
            cp.start()

    def wait_group(g, p):
        for cp in copies(g, p):
            cp.wait()

    def compute_group(g, p):
        rb = rbufs[p]
        eb = ebufs[p]

        def row_body(r, acc):
            base_row = r * F
            z = jnp.zeros((16,), jnp.float32)
            a0, a1, a2, a3, asq = z, z, z, z, z
            for f in range(F):
                x0 = rb[base_row + f, pl.ds(0, 16)]
                x1 = rb[base_row + f, pl.ds(16, 16)]
                x2 = rb[base_row + f, pl.ds(32, 16)]
                x3 = rb[base_row + f, pl.ds(48, 16)]
                asq = asq + x0 * x0 + x1 * x1 + x2 * x2 + x3 * x3
                a0, a1, a2, a3 = a0 + x0, a1 + x1, a2 + x2, a3 + x3
            t = a0 * a0 + a1 * a1 + a2 * a2 + a3 * a3 - asq
            e1a = eb[pl.ds(r * F, 16)]
            e1b = eb[pl.ds(r * F + 16, 16)]
            v = 0.5 * t + e1a + jnp.where(tail_mask, e1b, 0.0)
            s = jnp.sum(v)
            return jnp.where(lane == r, s, acc)

        acc = lax.fori_loop(0, GR, row_body, jnp.zeros((16,), jnp.float32))
        out_v[pl.ds(g * GR, GR)] = acc + bias_s

    start_group(0, 0)

    def body(i, _):
        g0 = 2 * i
        g1 = 2 * i + 1
        start_group(g1, 1)
        wait_group(g0, 0)
        compute_group(g0, 0)

        @pl.when(i < NG // 2 - 1)
        def _():
            start_group(g0 + 2, 0)

        wait_group(g1, 1)
        compute_group(g1, 1)
        return 0

    lax.fori_loop(0, NG // 2, body, 0)

    pltpu.sync_copy(out_v, out_hbm.at[pl.ds(base, BPW)])


@jax.jit
def _fm_sc(xi_flat, fm1_flat, fm_2nd, bias):
    mesh = plsc.VectorSubcoreMesh(core_axis_name="c", subcore_axis_name="s")
    fn = functools.partial(
        pl.kernel,
        mesh=mesh,
        compiler_params=pltpu.CompilerParams(
            needs_layout_passes=False, use_tc_tiling_on_sc=False),
        out_type=jax.ShapeDtypeStruct((B,), jnp.float32),
        scratch_types=[
            pltpu.VMEM((BPW * F,), jnp.int32),
            pltpu.VMEM((ROWS, D), jnp.float32),
            pltpu.VMEM((ROWS, D), jnp.float32),
            pltpu.VMEM((ROWS + 16,), jnp.float32),
            pltpu.VMEM((ROWS + 16,), jnp.float32),
            pltpu.VMEM((BPW,), jnp.float32),
            pltpu.VMEM((16,), jnp.float32),
            pltpu.SemaphoreType.DMA,
            pltpu.SemaphoreType.DMA,
        ],
    )(_fm_body)
    return fn(xi_flat, fm1_flat, fm_2nd, bias)


def kernel(xi, fm_1st, fm_2nd, bias):
    return _fm_sc(xi.reshape(-1), fm_1st.reshape(-1), fm_2nd, bias)
